# baseline (speedup 1.0000x reference)
import jax
import jax.numpy as jnp
from jax.experimental import pallas as pl
from jax.experimental.pallas import tpu as pltpu

N = 8192
R = 65
ONEHOT = 512
GLOB = 32
FEAT = 256
HID = 32
NH = 16
TFS = 64
C13 = 13

BLOCK = 128

_HIGH = jax.lax.Precision.HIGHEST


def _body(tfil_ref, tmask_ref, tedge_ref, tg_ref, toh_ref, tot_ref,
          Wemb_ref, Wsp_ref, Wse_ref, bse_ref, Woht_ref, Wefc_ref,
          Wl_ref, Wr_ref, Wtail_ref, out_ref,
          W2_s, WseB_s, selB_s, Wagg_s, Woh1_s, Woh2_s):
    i = pl.program_id(0)

    @pl.when(i == 0)
    def _fold():
        Wemb3 = Wemb_ref[...].reshape(ONEHOT, HID, HID)
        WspB = jnp.broadcast_to(Wsp_ref[...][None],
                                (ONEHOT, C13, HID))
        W2_3 = jax.lax.dot_general(
            Wemb3, WspB, (((1,), (2,)), ((0,), (0,))),
            precision=_HIGH,
            preferred_element_type=jnp.float32)
        W2_s[...] = W2_3.reshape(ONEHOT, HID * C13)
        WseB_s[...] = jnp.broadcast_to(Wse_ref[...][None],
                                       (BLOCK, HID, 3 * NH))
        r6 = jax.lax.broadcasted_iota(jnp.int32, (6, 96), 0)
        c96 = jax.lax.broadcasted_iota(jnp.int32, (6, 96), 1)
        one = jnp.float32(1.0)
        zero = jnp.float32(0.0)
        sel = jnp.where((c96 < 48) & (c96 // NH == r6), one, zero)
        sel = sel + jnp.where((c96 >= 48) & ((c96 - 48) // NH == r6 - 3),
                              one, zero)
        bse96 = jnp.concatenate(
            [jnp.zeros((1, 3 * NH), jnp.float32), bse_ref[...]], axis=1)
        sel = sel + jnp.where(
            (c96 >= 48) & ((c96 - 48) // NH == r6) & (r6 < 3),
            jnp.broadcast_to(bse96, (6, 96)), zero)
        selB_s[...] = jnp.broadcast_to(sel[None], (BLOCK, 6, 96))
        Wagg_s[...] = jnp.concatenate([
            jnp.dot(Wl_ref[...], Wtail_ref[HID:2 * HID],
                    precision=_HIGH, preferred_element_type=jnp.float32),
            jnp.dot(Wr_ref[...], Wtail_ref[2 * HID:3 * HID],
                    precision=_HIGH, preferred_element_type=jnp.float32),
            jnp.dot(Wefc_ref[...], Wtail_ref[0:HID],
                    precision=_HIGH, preferred_element_type=jnp.float32),
        ], axis=0)
        Woh = jnp.dot(Woht_ref[...], Wtail_ref[3 * HID:],
                      precision=_HIGH,
                      preferred_element_type=jnp.float32)
        Woh1_s[...] = Woh[:ONEHOT]
        Woh2_s[...] = Woh[ONEHOT:]

    toh = toh_ref[...]
    M = jnp.dot(toh, W2_s[...],
                preferred_element_type=jnp.float32)
    B = toh.shape[0]
    M3 = M.reshape(B, HID, C13)
    emb = jax.lax.dot_general(
        tot_ref[...], M3, (((2,), (2,)), ((0,), (0,))),
        preferred_element_type=jnp.float32)
    emb = jnp.maximum(emb, 0.0)
    emb2 = jax.lax.dot_general(
        emb, WseB_s[...], (((2,), (1,)), ((0,), (0,))),
        preferred_element_type=jnp.float32)

    big = jnp.float32(1e38)
    tfil = tfil_ref[...]
    tmask = tmask_ref[...]
    tedge = tedge_ref[...]
    tfa = -jnp.log(jnp.clip(1.0 - tfil * tmask, 1e-9, None))
    tfb = -jnp.log(jnp.clip(1.0 - (1.0 - tfil) * tmask, 1e-9, None))
    mb = (tmask - 1.0) * big
    eb = ((tedge > 0.0).astype(jnp.float32) - 1.0) * big
    aux3 = jnp.concatenate([tfa, tfb, tedge, mb, mb, eb],
                           axis=1).reshape(B, 6, R)
    sm = jax.lax.dot_general(
        aux3, selB_s[...], (((1,), (1,)), ((0,), (0,))),
        preferred_element_type=jnp.float32)
    mx = jnp.max(emb2 * sm[:, :, :48] + sm[:, :, 48:], axis=1)
    inv_big = jnp.float32(1e-38)
    m_any = 1.0 + jnp.max(mb, axis=1)[:, None] * inv_big
    e_any = 1.0 + jnp.max(eb, axis=1)[:, None] * inv_big
    anyv = jnp.concatenate([
        jnp.broadcast_to(m_any, (B, 2 * NH)),
        jnp.broadcast_to(e_any, (B, NH))], axis=1)
    agg = mx * anyv
    out = jnp.dot(agg, Wagg_s[...], preferred_element_type=jnp.float32)
    out = out + jnp.dot(toh, Woh1_s[...], preferred_element_type=jnp.float32)
    out = out + jnp.dot(tg_ref[...], Woh2_s[...],
                        preferred_element_type=jnp.float32)
    out_ref[...] = out


@jax.jit
def kernel(table_filter, table_mask, table_edge, table_global, table_onehot,
           table_others, W_onehot_emb, W_schema_prep, W_se, b_se,
           W_onehot_transform, W_edge_fc, W_left, W_right, W_tail):
    n = table_onehot.shape[0]
    grid = n // BLOCK
    out = pl.pallas_call(
        _body,
        grid=(grid,),
        in_specs=[
            pl.BlockSpec((BLOCK, R), lambda i: (i, 0)),
            pl.BlockSpec((BLOCK, R), lambda i: (i, 0)),
            pl.BlockSpec((BLOCK, R), lambda i: (i, 0)),
            pl.BlockSpec((BLOCK, GLOB), lambda i: (i, 0)),
            pl.BlockSpec((BLOCK, ONEHOT), lambda i: (i, 0)),
            pl.BlockSpec((BLOCK, R, C13), lambda i: (i, 0, 0)),
            pl.BlockSpec((ONEHOT, HID * HID), lambda i: (0, 0)),
            pl.BlockSpec((C13, HID), lambda i: (0, 0)),
            pl.BlockSpec((HID, 3 * NH), lambda i: (0, 0)),
            pl.BlockSpec((1, 3 * NH), lambda i: (0, 0)),
            pl.BlockSpec((ONEHOT + GLOB, FEAT), lambda i: (0, 0)),
            pl.BlockSpec((NH, HID), lambda i: (0, 0)),
            pl.BlockSpec((NH, HID), lambda i: (0, 0)),
            pl.BlockSpec((NH, HID), lambda i: (0, 0)),
            pl.BlockSpec((3 * HID + FEAT, FEAT), lambda i: (0, 0)),
        ],
        out_specs=pl.BlockSpec((BLOCK, FEAT), lambda i: (i, 0)),
        out_shape=jax.ShapeDtypeStruct((n, FEAT), jnp.float32),
        scratch_shapes=[
            pltpu.VMEM((ONEHOT, HID * C13), jnp.float32),
            pltpu.VMEM((BLOCK, HID, 3 * NH), jnp.float32),
            pltpu.VMEM((BLOCK, 6, 96), jnp.float32),
            pltpu.VMEM((3 * NH, FEAT), jnp.float32),
            pltpu.VMEM((ONEHOT, FEAT), jnp.float32),
            pltpu.VMEM((GLOB, FEAT), jnp.float32),
        ],
    )(table_filter, table_mask, table_edge, table_global, table_onehot,
      table_others, W_onehot_emb, W_schema_prep, W_se,
      b_se.reshape(1, 3 * NH), W_onehot_transform, W_edge_fc,
      W_left, W_right, W_tail)
    return out


# speedup vs baseline: 1.0006x; 1.0006x over previous
import jax
import jax.numpy as jnp
from jax.experimental import pallas as pl
from jax.experimental.pallas import tpu as pltpu

N = 8192
R = 65
ONEHOT = 512
GLOB = 32
FEAT = 256
HID = 32
NH = 16
TFS = 64
C13 = 13

BLOCK = 128

_HIGH = jax.lax.Precision.HIGHEST


def _body(tfil_ref, tmask_ref, tedge_ref, tg_ref, toh_ref, tot_ref,
          Wemb_ref, Wsp_ref, Wse_ref, bse_ref, Woht_ref, Wefc_ref,
          Wl_ref, Wr_ref, Wtail_ref, out_ref,
          W2_s, WseB_s, selB_s, Wagg_s, Woh1_s, Woh2_s):
    i = pl.program_id(0)

    @pl.when(i == 0)
    def _fold():
        Wemb3 = Wemb_ref[...].reshape(ONEHOT, HID, HID)
        WspB = jnp.broadcast_to(Wsp_ref[...][None],
                                (ONEHOT, C13, HID))
        W2_3 = jax.lax.dot_general(
            Wemb3, WspB, (((1,), (2,)), ((0,), (0,))),
            precision=_HIGH,
            preferred_element_type=jnp.float32)
        W2_s[...] = W2_3.reshape(ONEHOT, HID * C13).astype(jnp.bfloat16)
        WseB_s[...] = jnp.broadcast_to(Wse_ref[...][None],
                                       (BLOCK, HID, 3 * NH))
        r6 = jax.lax.broadcasted_iota(jnp.int32, (6, 96), 0)
        c96 = jax.lax.broadcasted_iota(jnp.int32, (6, 96), 1)
        one = jnp.float32(1.0)
        zero = jnp.float32(0.0)
        sel = jnp.where((c96 < 48) & (c96 // NH == r6), one, zero)
        sel = sel + jnp.where((c96 >= 48) & ((c96 - 48) // NH == r6 - 3),
                              one, zero)
        bse96 = jnp.concatenate(
            [jnp.zeros((1, 3 * NH), jnp.float32), bse_ref[...]], axis=1)
        sel = sel + jnp.where(
            (c96 >= 48) & ((c96 - 48) // NH == r6) & (r6 < 3),
            jnp.broadcast_to(bse96, (6, 96)), zero)
        selB_s[...] = jnp.broadcast_to(sel[None], (BLOCK, 6, 96))
        Wagg_s[...] = jnp.concatenate([
            jnp.dot(Wl_ref[...], Wtail_ref[HID:2 * HID],
                    precision=_HIGH, preferred_element_type=jnp.float32),
            jnp.dot(Wr_ref[...], Wtail_ref[2 * HID:3 * HID],
                    precision=_HIGH, preferred_element_type=jnp.float32),
            jnp.dot(Wefc_ref[...], Wtail_ref[0:HID],
                    precision=_HIGH, preferred_element_type=jnp.float32),
        ], axis=0)
        Woh = jnp.dot(Woht_ref[...], Wtail_ref[3 * HID:],
                      precision=_HIGH,
                      preferred_element_type=jnp.float32)
        Woh1_s[...] = Woh[:ONEHOT].astype(jnp.bfloat16)
        Woh2_s[...] = Woh[ONEHOT:]

    toh = toh_ref[...]
    M = jnp.dot(toh, W2_s[...],
                preferred_element_type=jnp.float32)
    B = toh.shape[0]
    M3 = M.reshape(B, HID, C13)
    emb = jax.lax.dot_general(
        tot_ref[...], M3.astype(jnp.bfloat16),
        (((2,), (2,)), ((0,), (0,))),
        preferred_element_type=jnp.float32)
    emb = jnp.maximum(emb, 0.0)
    emb2 = jax.lax.dot_general(
        emb, WseB_s[...], (((2,), (1,)), ((0,), (0,))),
        preferred_element_type=jnp.float32)

    big = jnp.float32(1e38)
    tfil = tfil_ref[...]
    tmask = tmask_ref[...]
    tedge = tedge_ref[...]
    tfa = -jnp.log(jnp.clip(1.0 - tfil * tmask, 1e-9, None))
    tfb = -jnp.log(jnp.clip(1.0 - (1.0 - tfil) * tmask, 1e-9, None))
    mb = (tmask - 1.0) * big
    eb = ((tedge > 0.0).astype(jnp.float32) - 1.0) * big
    aux3 = jnp.concatenate([tfa, tfb, tedge, mb, mb, eb],
                           axis=1).reshape(B, 6, R)
    sm = jax.lax.dot_general(
        aux3, selB_s[...], (((1,), (1,)), ((0,), (0,))),
        preferred_element_type=jnp.float32)
    mx = jnp.max(emb2 * sm[:, :, :48] + sm[:, :, 48:], axis=1)
    inv_big = jnp.float32(1e-38)
    m_any = 1.0 + jnp.max(mb, axis=1)[:, None] * inv_big
    e_any = 1.0 + jnp.max(eb, axis=1)[:, None] * inv_big
    anyv = jnp.concatenate([
        jnp.broadcast_to(m_any, (B, 2 * NH)),
        jnp.broadcast_to(e_any, (B, NH))], axis=1)
    agg = mx * anyv
    out = jnp.dot(agg, Wagg_s[...], preferred_element_type=jnp.float32)
    out = out + jnp.dot(toh, Woh1_s[...],
                        preferred_element_type=jnp.float32)
    out = out + jnp.dot(tg_ref[...], Woh2_s[...],
                        preferred_element_type=jnp.float32)
    out_ref[...] = out


@jax.jit
def kernel(table_filter, table_mask, table_edge, table_global, table_onehot,
           table_others, W_onehot_emb, W_schema_prep, W_se, b_se,
           W_onehot_transform, W_edge_fc, W_left, W_right, W_tail):
    n = table_onehot.shape[0]
    grid = n // BLOCK
    out = pl.pallas_call(
        _body,
        grid=(grid,),
        in_specs=[
            pl.BlockSpec((BLOCK, R), lambda i: (i, 0)),
            pl.BlockSpec((BLOCK, R), lambda i: (i, 0)),
            pl.BlockSpec((BLOCK, R), lambda i: (i, 0)),
            pl.BlockSpec((BLOCK, GLOB), lambda i: (i, 0)),
            pl.BlockSpec((BLOCK, ONEHOT), lambda i: (i, 0)),
            pl.BlockSpec((BLOCK, R, C13), lambda i: (i, 0, 0)),
            pl.BlockSpec((ONEHOT, HID * HID), lambda i: (0, 0)),
            pl.BlockSpec((C13, HID), lambda i: (0, 0)),
            pl.BlockSpec((HID, 3 * NH), lambda i: (0, 0)),
            pl.BlockSpec((1, 3 * NH), lambda i: (0, 0)),
            pl.BlockSpec((ONEHOT + GLOB, FEAT), lambda i: (0, 0)),
            pl.BlockSpec((NH, HID), lambda i: (0, 0)),
            pl.BlockSpec((NH, HID), lambda i: (0, 0)),
            pl.BlockSpec((NH, HID), lambda i: (0, 0)),
            pl.BlockSpec((3 * HID + FEAT, FEAT), lambda i: (0, 0)),
        ],
        out_specs=pl.BlockSpec((BLOCK, FEAT), lambda i: (i, 0)),
        out_shape=jax.ShapeDtypeStruct((n, FEAT), jnp.float32),
        scratch_shapes=[
            pltpu.VMEM((ONEHOT, HID * C13), jnp.bfloat16),
            pltpu.VMEM((BLOCK, HID, 3 * NH), jnp.float32),
            pltpu.VMEM((BLOCK, 6, 96), jnp.float32),
            pltpu.VMEM((3 * NH, FEAT), jnp.float32),
            pltpu.VMEM((ONEHOT, FEAT), jnp.bfloat16),
            pltpu.VMEM((GLOB, FEAT), jnp.float32),
        ],
    )(table_filter, table_mask, table_edge, table_global,
      table_onehot.astype(jnp.bfloat16),
      table_others.astype(jnp.bfloat16), W_onehot_emb, W_schema_prep, W_se,
      b_se.reshape(1, 3 * NH), W_onehot_transform, W_edge_fc,
      W_left, W_right, W_tail)
    return out
